# baseline (speedup 1.0000x reference)
import functools

import jax
import jax.numpy as jnp
from jax import lax
from jax.experimental import pallas as pl
from jax.experimental.pallas import tpu as pltpu
from jax.experimental.pallas import tpu_sc as plsc

N = 100000
E1 = 800000
E2 = 800000
E = E1 + E2
B = 64
NW = 32
PER_W = E // NW
CHUNK = 4992
NDMA = CHUNK // 128
NCH = PER_W // CHUNK
TAIL = PER_W - NCH * CHUNK
PT_WORDS = N // 4
NODE_W = 25
NODE_PER_W = N // NODE_W
CCH = 2000
NCC = PER_W // CCH

_mesh = plsc.VectorSubcoreMesh(core_axis_name="c", subcore_axis_name="s")
_params = pltpu.CompilerParams(
    needs_layout_passes=False, use_tc_tiling_on_sc=False)

_iota16 = lambda: lax.iota(jnp.int32, 16)


def _worker_id():
    return lax.axis_index("c") * 16 + lax.axis_index("s")


def _hist_update(hist, key):
    cnt, last = plsc.scan_count(key)
    h = plsc.load_gather(hist, [key])
    plsc.store_scatter(hist, [key], h + cnt, mask=last)


@functools.partial(
    pl.kernel,
    out_type=(
        jax.ShapeDtypeStruct((NW * B,), jnp.int32),
        jax.ShapeDtypeStruct((NW * B,), jnp.int32),
        jax.ShapeDtypeStruct((E,), jnp.int32),
    ),
    mesh=_mesh,
    compiler_params=_params,
    scratch_types=[
        pltpu.VMEM((PT_WORDS,), jnp.int32),
        pltpu.VMEM((CHUNK,), jnp.int32),
        pltpu.VMEM((CHUNK,), jnp.int32),
        pltpu.VMEM((B,), jnp.int32),
        pltpu.VMEM((B,), jnp.int32),
        pltpu.VMEM((NODE_PER_W,), jnp.int32),
    ],
)
def _pass_a(src_hbm, ptable_hbm, n2g_hbm,
            hist_e_hbm, hist_n_hbm, keys_hbm,
            ptable, srcbuf, keybuf, hist_e, hist_n, nodebuf):
    wid = _worker_id()
    start = wid * PER_W
    pltpu.sync_copy(ptable_hbm, ptable)
    for j in range(B // 16):
        z = jnp.zeros((16,), jnp.int32)
        hist_e[pl.ds(j * 16, 16)] = z
        hist_n[pl.ds(j * 16, 16)] = z

    def do_vec(i, _):
        src = srcbuf[pl.ds(i * 16, 16)]
        word = plsc.load_gather(ptable, [lax.shift_right_logical(src, 2)])
        key = lax.shift_right_logical(word, (src & 3) * 8) & 255
        keybuf[pl.ds(i * 16, 16)] = key
        _hist_update(hist_e, key)
        return 0

    def load_src(cstart, n):
        pltpu.sync_copy(src_hbm.at[pl.ds(cstart, n)], srcbuf.at[pl.ds(0, n)])

    def do_chunk(k, _):
        load_src(start + k * CHUNK, CHUNK)
        lax.fori_loop(0, CHUNK // 16, do_vec, 0)
        pltpu.sync_copy(
            keybuf, keys_hbm.at[pl.ds(wid * PER_W + k * CHUNK, CHUNK)])
        return 0

    lax.fori_loop(0, NCH, do_chunk, 0)

    load_src(start + NCH * CHUNK, TAIL)
    lax.fori_loop(0, TAIL // 16, do_vec, 0)
    pltpu.sync_copy(keybuf.at[pl.ds(0, TAIL)],
                    keys_hbm.at[pl.ds(wid * PER_W + NCH * CHUNK, TAIL)])
    pltpu.sync_copy(hist_e, hist_e_hbm.at[pl.ds(wid * B, B)])

    @pl.when(wid < NODE_W)
    def _():
        pltpu.sync_copy(n2g_hbm.at[pl.ds(wid * NODE_PER_W, NODE_PER_W)],
                        nodebuf)

        def do_nvec(i, _):
            _hist_update(hist_n, nodebuf[pl.ds(i * 16, 16)])
            return 0

        lax.fori_loop(0, NODE_PER_W // 16, do_nvec, 0)

    pltpu.sync_copy(hist_n, hist_n_hbm.at[pl.ds(wid * B, B)])


@functools.partial(
    pl.kernel,
    out_type=(
        jax.ShapeDtypeStruct((E, 8), jnp.int32),
        jax.ShapeDtypeStruct((B,), jnp.int32),
    ),
    mesh=_mesh,
    compiler_params=_params,
    scratch_types=[
        pltpu.VMEM((CHUNK,), jnp.int32),
        pltpu.VMEM((CHUNK,), jnp.int32),
        pltpu.VMEM((CHUNK,), jnp.int32),
        pltpu.VMEM((CHUNK,), jnp.int32),
        pltpu.VMEM((2, CHUNK, 8), jnp.int32),
        pltpu.VMEM((2, NDMA, 128), jnp.int32),
        pltpu.VMEM((TAIL,), jnp.int32),
        pltpu.VMEM((NW * B,), jnp.int32),
        pltpu.VMEM((B,), jnp.int32),
        pltpu.VMEM((B,), jnp.int32),
        pltpu.VMEM((B,), jnp.int32),
        pltpu.SemaphoreType.DMA,
        pltpu.SemaphoreType.DMA,
    ],
)
def _pass_b(src_hbm, dst_hbm, rel_hbm, keys_hbm,
            hist_e_hbm, hist_n_hbm,
            stage_hbm, nedges_hbm,
            srcb, dstb, relb, keybuf, stage2, pos2d2, postail,
            histbuf, base, noff, nedge, semA, semB):
    wid = _worker_id()
    start = wid * PER_W

    pltpu.sync_copy(hist_e_hbm, histbuf)
    for j in range(B // 16):
        acc = jnp.zeros((16,), jnp.int32)
        mine = jnp.zeros((16,), jnp.int32)
        for w in range(NW):
            row = histbuf[pl.ds(w * B + j * 16, 16)]
            acc = acc + row
            wv = jnp.full((16,), w, jnp.int32)
            mine = mine + jnp.where(wv < wid, row, 0)
        nedge[pl.ds(j * 16, 16)] = acc
        base[pl.ds(j * 16, 16)] = mine

    @pl.when(wid == 0)
    def _():
        pltpu.sync_copy(nedge, nedges_hbm)

    carry = jnp.zeros((), jnp.int32)
    for j in range(B // 16):
        v = nedge[pl.ds(j * 16, 16)]
        inc = plsc.cumsum(v)
        base[pl.ds(j * 16, 16)] = base[pl.ds(j * 16, 16)] + inc - v + carry
        carry = carry + jnp.sum(v)

    pltpu.sync_copy(hist_n_hbm, histbuf)
    carry = jnp.zeros((), jnp.int32)
    for j in range(B // 16):
        acc = jnp.zeros((16,), jnp.int32)
        for w in range(NODE_W):
            acc = acc + histbuf[pl.ds(w * B + j * 16, 16)]
        inc = plsc.cumsum(acc)
        noff[pl.ds(j * 16, 16)] = inc - acc + carry
        carry = carry + jnp.sum(acc)

    c0 = jnp.zeros((16,), jnp.int32)
    c1 = jnp.full((16,), 1, jnp.int32)
    c2 = jnp.full((16,), 2, jnp.int32)
    c3 = jnp.full((16,), 3, jnp.int32)

    def make_place_vec(stage):
        def place_vec(i):
            rowi = _iota16() + i * 16
            sl = pl.ds(i * 16, 16)
            key = keybuf[sl]
            cnt, last = plsc.scan_count(key)
            b = plsc.load_gather(base, [key])
            pos = b + cnt - 1
            plsc.store_scatter(base, [key], b + cnt, mask=last)
            plsc.store_scatter(stage, [rowi, c0], srcb[sl])
            plsc.store_scatter(stage, [rowi, c1], dstb[sl])
            plsc.store_scatter(stage, [rowi, c2], relb[sl])
            plsc.store_scatter(stage, [rowi, c3],
                               plsc.load_gather(noff, [key]))
            return pos
        return place_vec

    def load_cols(cstart, n):
        pltpu.sync_copy(src_hbm.at[pl.ds(cstart, n)], srcb.at[pl.ds(0, n)])
        pltpu.sync_copy(dst_hbm.at[pl.ds(cstart, n)], dstb.at[pl.ds(0, n)])
        pltpu.sync_copy(rel_hbm.at[pl.ds(cstart, n)], relb.at[pl.ds(0, n)])

    pending = [None, None]
    for k in range(NCH):
        p = k % 2
        stage = stage2.at[p]
        sem = semA if p == 0 else semB
        if pending[p] is not None:
            for cp in pending[p]:
                cp.wait()
        load_cols(start + k * CHUNK, CHUNK)
        pltpu.sync_copy(keys_hbm.at[pl.ds(wid * PER_W + k * CHUNK, CHUNK)],
                        keybuf)
        place_vec = make_place_vec(stage)

        def do_vec(i, _, place_vec=place_vec, p=p):
            pos = place_vec(i)
            r = i // 8
            col = (i % 8) * 16
            plsc.store_scatter(pos2d2.at[p],
                               [jnp.full((16,), r, jnp.int32),
                                col + _iota16()], pos)
            return 0

        lax.fori_loop(0, CHUNK // 16, do_vec, 0)
        pending[p] = [pltpu.async_copy(stage.at[pl.ds(j * 128, 128)],
                                       stage_hbm.at[pos2d2.at[p, j]], sem)
                      for j in range(NDMA)]

    for p in (0, 1):
        if pending[p] is not None:
            for cp in pending[p]:
                cp.wait()
    load_cols(start + NCH * CHUNK, TAIL)
    pltpu.sync_copy(keys_hbm.at[pl.ds(wid * PER_W + NCH * CHUNK, TAIL)],
                    keybuf.at[pl.ds(0, TAIL)])
    tail_place = make_place_vec(stage2.at[0])

    def do_tail_vec(i, _):
        postail[pl.ds(i * 16, 16)] = tail_place(i)
        return 0

    lax.fori_loop(0, TAIL // 16, do_tail_vec, 0)
    pltpu.async_copy(stage2.at[0, pl.ds(0, TAIL)],
                     stage_hbm.at[postail], semA).wait()


@functools.partial(
    pl.kernel,
    out_type=(
        jax.ShapeDtypeStruct((3, E), jnp.int32),
        jax.ShapeDtypeStruct((E,), jnp.int32),
        jax.ShapeDtypeStruct((E,), jnp.float32),
    ),
    mesh=_mesh,
    compiler_params=_params,
    scratch_types=[
        pltpu.VMEM((2, CCH, 8), jnp.int32),
        pltpu.VMEM((2, CCH), jnp.int32),
        pltpu.VMEM((2, CCH), jnp.int32),
        pltpu.VMEM((2, CCH), jnp.int32),
        pltpu.VMEM((2, CCH), jnp.int32),
        pltpu.VMEM((CCH,), jnp.float32),
        pltpu.SemaphoreType.DMA,
        pltpu.SemaphoreType.DMA,
        pltpu.SemaphoreType.DMA,
        pltpu.SemaphoreType.DMA,
    ],
)
def _pass_c(stage_hbm, edge3_hbm, offs_hbm, ones_hbm,
            rows2, sb2, db2, rb2, ob2, onesb, semI0, semI1, semO0, semO1):
    wid = _worker_id()
    start = wid * PER_W
    c0 = jnp.zeros((16,), jnp.int32)
    c1 = jnp.full((16,), 1, jnp.int32)
    c2 = jnp.full((16,), 2, jnp.int32)
    c3 = jnp.full((16,), 3, jnp.int32)
    semI = (semI0, semI1)
    semO = (semO0, semO1)

    def fill_ones(i, _):
        onesb[pl.ds(i * 16, 16)] = jnp.ones((16,), jnp.float32)
        return 0

    lax.fori_loop(0, CCH // 16, fill_ones, 0)

    def issue_in(k):
        p = k % 2
        return pltpu.async_copy(
            stage_hbm.at[pl.ds(start + k * CCH, CCH)], rows2.at[p], semI[p])

    pend_in = {0: issue_in(0)}
    pend_out = [None, None]
    for k in range(NCC):
        p = k % 2
        pend_in.pop(k).wait()
        if k + 1 < NCC:
            pend_in[k + 1] = issue_in(k + 1)
        if pend_out[p] is not None:
            for cp in pend_out[p]:
                cp.wait()
        rows = rows2.at[p]
        sb, db, rb, ob = sb2.at[p], db2.at[p], rb2.at[p], ob2.at[p]

        def do_vec(i, _, rows=rows, sb=sb, db=db, rb=rb, ob=ob):
            rowi = _iota16() + i * 16
            sl = pl.ds(i * 16, 16)
            sb[sl] = plsc.load_gather(rows, [rowi, c0])
            db[sl] = plsc.load_gather(rows, [rowi, c1])
            rb[sl] = plsc.load_gather(rows, [rowi, c2])
            ob[sl] = plsc.load_gather(rows, [rowi, c3])
            return 0

        lax.fori_loop(0, CCH // 16, do_vec, 0)
        out_sl = pl.ds(start + k * CCH, CCH)
        pend_out[p] = [
            pltpu.async_copy(sb, edge3_hbm.at[0, out_sl], semO[p]),
            pltpu.async_copy(db, edge3_hbm.at[1, out_sl], semO[p]),
            pltpu.async_copy(rb, edge3_hbm.at[2, out_sl], semO[p]),
            pltpu.async_copy(ob, offs_hbm.at[out_sl], semO[p]),
            pltpu.async_copy(onesb, ones_hbm.at[out_sl], semO[p]),
        ]
    for p in (0, 1):
        if pend_out[p] is not None:
            for cp in pend_out[p]:
                cp.wait()


def kernel(x, edges1, edges2, node2graph, num_relation1, num_relation2):
    ptable = lax.bitcast_convert_type(
        node2graph.astype(jnp.int8).reshape(PT_WORDS, 4), jnp.int32)
    src = jnp.concatenate([edges1[:, 0], edges2[:, 0]])
    dst = jnp.concatenate([edges1[:, 1], edges2[:, 1]])
    rel = jnp.concatenate(
        [edges1[:, 2], edges2[:, 2] + jnp.asarray(num_relation1, jnp.int32)])
    hist_e, hist_n, keys_all = _pass_a(src, ptable, node2graph)
    staging, num_edges = _pass_b(src, dst, rel, keys_all, hist_e, hist_n)
    edge3, offsets, edge_weight = _pass_c(staging)
    edge_list = edge3.T
    return (x, edge_list, edge_weight, num_edges, offsets)


# speedup vs baseline: 1.5883x; 1.5883x over previous
import functools

import jax
import jax.numpy as jnp
from jax import lax
from jax.experimental import pallas as pl
from jax.experimental.pallas import tpu as pltpu
from jax.experimental.pallas import tpu_sc as plsc

N = 100000
E1 = 800000
E2 = 800000
E = E1 + E2
B = 64
NW = 32
PER_W = E // NW
CHUNK = 4992
NDMA = CHUNK // 128
NCH = PER_W // CHUNK
TAIL = PER_W - NCH * CHUNK
PT_WORDS = N // 4
NODE_W = 25
NODE_PER_W = N // NODE_W
CCH = 2000
NCC = PER_W // CCH

_mesh = plsc.VectorSubcoreMesh(core_axis_name="c", subcore_axis_name="s")
_params = pltpu.CompilerParams(
    needs_layout_passes=False, use_tc_tiling_on_sc=False)

_iota16 = lambda: lax.iota(jnp.int32, 16)


def _worker_id():
    return lax.axis_index("c") * 16 + lax.axis_index("s")


def _hist_update(hist, key):
    cnt, last = plsc.scan_count(key)
    h = plsc.load_gather(hist, [key])
    plsc.store_scatter(hist, [key], h + cnt, mask=last)


@functools.partial(
    pl.kernel,
    out_type=(
        jax.ShapeDtypeStruct((NW * B,), jnp.int32),
        jax.ShapeDtypeStruct((NW * B,), jnp.int32),
        jax.ShapeDtypeStruct((E,), jnp.int32),
    ),
    mesh=_mesh,
    compiler_params=_params,
    scratch_types=[
        pltpu.VMEM((PT_WORDS,), jnp.int32),
        pltpu.VMEM((2, CHUNK), jnp.int32),
        pltpu.VMEM((2, CHUNK), jnp.int32),
        pltpu.VMEM((4 * B,), jnp.int32),
        pltpu.VMEM((B,), jnp.int32),
        pltpu.VMEM((B,), jnp.int32),
        pltpu.VMEM((NODE_PER_W,), jnp.int32),
        pltpu.SemaphoreType.DMA,
        pltpu.SemaphoreType.DMA,
        pltpu.SemaphoreType.DMA,
        pltpu.SemaphoreType.DMA,
    ],
)
def _pass_a(src_hbm, ptable_hbm, n2g_hbm,
            hist_e_hbm, hist_n_hbm, keys_hbm,
            ptable, srcbuf2, keybuf2, hist4, hist_e, hist_n, nodebuf,
            semI0, semI1, semK0, semK1):
    wid = _worker_id()
    start = wid * PER_W
    semI = (semI0, semI1)
    semK = (semK0, semK1)
    pltpu.sync_copy(ptable_hbm, ptable)
    z = jnp.zeros((16,), jnp.int32)
    for j in range(4 * B // 16):
        hist4[pl.ds(j * 16, 16)] = z
    for j in range(B // 16):
        hist_n[pl.ds(j * 16, 16)] = z

    def make_do_group(sb, kb):
        def do_group(g, _):
            for q in range(4):
                i = g * 4 + q
                sl = pl.ds(i * 16, 16)
                src = sb[sl]
                word = plsc.load_gather(
                    ptable, [lax.shift_right_logical(src, 2)])
                key = lax.shift_right_logical(word, (src & 3) * 8) & 255
                kb[sl] = key
                cnt, last = plsc.scan_count(key)
                hidx = key + q * B
                h = plsc.load_gather(hist4, [hidx])
                plsc.store_scatter(hist4, [hidx], h + cnt, mask=last)
            return 0
        return do_group

    def issue_in(k):
        p = k % 2
        return pltpu.async_copy(
            src_hbm.at[pl.ds(start + k * CHUNK, CHUNK)],
            srcbuf2.at[p], semI[p])

    pend_in = {0: issue_in(0)}
    pend_key = [None, None]
    for k in range(NCH):
        p = k % 2
        pend_in.pop(k).wait()
        if k + 1 < NCH:
            pend_in[k + 1] = issue_in(k + 1)
        if pend_key[p] is not None:
            pend_key[p].wait()
        lax.fori_loop(0, CHUNK // 64,
                      make_do_group(srcbuf2.at[p], keybuf2.at[p]), 0)
        pend_key[p] = pltpu.async_copy(
            keybuf2.at[p],
            keys_hbm.at[pl.ds(wid * PER_W + k * CHUNK, CHUNK)], semK[p])
    for p in (0, 1):
        if pend_key[p] is not None:
            pend_key[p].wait()

    pltpu.sync_copy(src_hbm.at[pl.ds(start + NCH * CHUNK, TAIL)],
                    srcbuf2.at[0, pl.ds(0, TAIL)])

    def do_tail_vec(i, _):
        src = srcbuf2.at[0][pl.ds(i * 16, 16)]
        word = plsc.load_gather(ptable, [lax.shift_right_logical(src, 2)])
        key = lax.shift_right_logical(word, (src & 3) * 8) & 255
        keybuf2.at[0][pl.ds(i * 16, 16)] = key
        _hist_update(hist4, key)
        return 0

    lax.fori_loop(0, TAIL // 16, do_tail_vec, 0)
    pltpu.sync_copy(keybuf2.at[0, pl.ds(0, TAIL)],
                    keys_hbm.at[pl.ds(wid * PER_W + NCH * CHUNK, TAIL)])
    for j in range(B // 16):
        s = (hist4[pl.ds(j * 16, 16)] + hist4[pl.ds(B + j * 16, 16)]
             + hist4[pl.ds(2 * B + j * 16, 16)]
             + hist4[pl.ds(3 * B + j * 16, 16)])
        hist_e[pl.ds(j * 16, 16)] = s
    pltpu.sync_copy(hist_e, hist_e_hbm.at[pl.ds(wid * B, B)])

    @pl.when(wid < NODE_W)
    def _():
        pltpu.sync_copy(n2g_hbm.at[pl.ds(wid * NODE_PER_W, NODE_PER_W)],
                        nodebuf)

        def do_nvec(i, _):
            _hist_update(hist_n, nodebuf[pl.ds(i * 16, 16)])
            return 0

        lax.fori_loop(0, NODE_PER_W // 16, do_nvec, 0)

    pltpu.sync_copy(hist_n, hist_n_hbm.at[pl.ds(wid * B, B)])


@functools.partial(
    pl.kernel,
    out_type=(
        jax.ShapeDtypeStruct((E, 8), jnp.int32),
        jax.ShapeDtypeStruct((B,), jnp.int32),
    ),
    mesh=_mesh,
    compiler_params=_params,
    scratch_types=[
        pltpu.VMEM((CHUNK,), jnp.int32),
        pltpu.VMEM((CHUNK,), jnp.int32),
        pltpu.VMEM((CHUNK,), jnp.int32),
        pltpu.VMEM((CHUNK,), jnp.int32),
        pltpu.VMEM((2, CHUNK, 8), jnp.int32),
        pltpu.VMEM((2, NDMA, 128), jnp.int32),
        pltpu.VMEM((CHUNK,), jnp.int32),
        pltpu.VMEM((CHUNK,), jnp.int32),
        pltpu.VMEM((TAIL,), jnp.int32),
        pltpu.VMEM((NW * B,), jnp.int32),
        pltpu.VMEM((B,), jnp.int32),
        pltpu.VMEM((B,), jnp.int32),
        pltpu.VMEM((B,), jnp.int32),
        pltpu.SemaphoreType.DMA,
        pltpu.SemaphoreType.DMA,
    ],
)
def _pass_b(src_hbm, dst_hbm, rel_hbm, keys_hbm,
            hist_e_hbm, hist_n_hbm,
            stage_hbm, nedges_hbm,
            srcb, dstb, relb, keybuf, stage2, pos2d2, cntbuf, lastbuf, postail,
            histbuf, base, noff, nedge, semA, semB):
    wid = _worker_id()
    start = wid * PER_W

    pltpu.sync_copy(hist_e_hbm, histbuf)
    for j in range(B // 16):
        acc = jnp.zeros((16,), jnp.int32)
        mine = jnp.zeros((16,), jnp.int32)
        for w in range(NW):
            row = histbuf[pl.ds(w * B + j * 16, 16)]
            acc = acc + row
            wv = jnp.full((16,), w, jnp.int32)
            mine = mine + jnp.where(wv < wid, row, 0)
        nedge[pl.ds(j * 16, 16)] = acc
        base[pl.ds(j * 16, 16)] = mine

    @pl.when(wid == 0)
    def _():
        pltpu.sync_copy(nedge, nedges_hbm)

    carry = jnp.zeros((), jnp.int32)
    for j in range(B // 16):
        v = nedge[pl.ds(j * 16, 16)]
        inc = plsc.cumsum(v)
        base[pl.ds(j * 16, 16)] = base[pl.ds(j * 16, 16)] + inc - v + carry
        carry = carry + jnp.sum(v)

    pltpu.sync_copy(hist_n_hbm, histbuf)
    carry = jnp.zeros((), jnp.int32)
    for j in range(B // 16):
        acc = jnp.zeros((16,), jnp.int32)
        for w in range(NODE_W):
            acc = acc + histbuf[pl.ds(w * B + j * 16, 16)]
        inc = plsc.cumsum(acc)
        noff[pl.ds(j * 16, 16)] = inc - acc + carry
        carry = carry + jnp.sum(acc)

    c0 = jnp.zeros((16,), jnp.int32)
    c1 = jnp.full((16,), 1, jnp.int32)
    c2 = jnp.full((16,), 2, jnp.int32)
    c3 = jnp.full((16,), 3, jnp.int32)

    def make_scan_vec(stage):
        def scan_vec(i, _):
            rowi = _iota16() + i * 16
            sl = pl.ds(i * 16, 16)
            key = keybuf[sl]
            cnt, last = plsc.scan_count(key)
            cntbuf[sl] = cnt
            lastbuf[sl] = jnp.where(last, 1, 0).astype(jnp.int32)
            plsc.store_scatter(stage, [rowi, c0], srcb[sl])
            plsc.store_scatter(stage, [rowi, c1], dstb[sl])
            plsc.store_scatter(stage, [rowi, c2], relb[sl])
            plsc.store_scatter(stage, [rowi, c3],
                               plsc.load_gather(noff, [key]))
            return 0
        return scan_vec

    def place_vec(i):
        sl = pl.ds(i * 16, 16)
        key = keybuf[sl]
        cnt = cntbuf[sl]
        last = lastbuf[sl] > 0
        b = plsc.load_gather(base, [key])
        pos = b + cnt - 1
        plsc.store_scatter(base, [key], b + cnt, mask=last)
        return pos

    def load_cols(cstart, n):
        pltpu.sync_copy(src_hbm.at[pl.ds(cstart, n)], srcb.at[pl.ds(0, n)])
        pltpu.sync_copy(dst_hbm.at[pl.ds(cstart, n)], dstb.at[pl.ds(0, n)])
        pltpu.sync_copy(rel_hbm.at[pl.ds(cstart, n)], relb.at[pl.ds(0, n)])

    pending = [None, None]
    for k in range(NCH):
        p = k % 2
        stage = stage2.at[p]
        sem = semA if p == 0 else semB
        if pending[p] is not None:
            for cp in pending[p]:
                cp.wait()
        load_cols(start + k * CHUNK, CHUNK)
        pltpu.sync_copy(keys_hbm.at[pl.ds(wid * PER_W + k * CHUNK, CHUNK)],
                        keybuf)
        lax.fori_loop(0, CHUNK // 16, make_scan_vec(stage), 0)

        def do_vec(i, _, p=p):
            pos = place_vec(i)
            r = i // 8
            col = (i % 8) * 16
            plsc.store_scatter(pos2d2.at[p],
                               [jnp.full((16,), r, jnp.int32),
                                col + _iota16()], pos)
            return 0

        lax.fori_loop(0, CHUNK // 16, do_vec, 0)
        pending[p] = [pltpu.async_copy(stage.at[pl.ds(j * 128, 128)],
                                       stage_hbm.at[pos2d2.at[p, j]], sem)
                      for j in range(NDMA)]

    for p in (0, 1):
        if pending[p] is not None:
            for cp in pending[p]:
                cp.wait()
    load_cols(start + NCH * CHUNK, TAIL)
    pltpu.sync_copy(keys_hbm.at[pl.ds(wid * PER_W + NCH * CHUNK, TAIL)],
                    keybuf.at[pl.ds(0, TAIL)])
    lax.fori_loop(0, TAIL // 16, make_scan_vec(stage2.at[0]), 0)

    def do_tail_vec(i, _):
        postail[pl.ds(i * 16, 16)] = place_vec(i)
        return 0

    lax.fori_loop(0, TAIL // 16, do_tail_vec, 0)
    pltpu.async_copy(stage2.at[0, pl.ds(0, TAIL)],
                     stage_hbm.at[postail], semA).wait()


@functools.partial(
    pl.kernel,
    out_type=(
        jax.ShapeDtypeStruct((E,), jnp.int32),
        jax.ShapeDtypeStruct((E,), jnp.int32),
        jax.ShapeDtypeStruct((E,), jnp.int32),
        jax.ShapeDtypeStruct((E,), jnp.int32),
        jax.ShapeDtypeStruct((E,), jnp.float32),
    ),
    mesh=_mesh,
    compiler_params=_params,
    scratch_types=[
        pltpu.VMEM((2, CCH, 8), jnp.int32),
        pltpu.VMEM((2, CCH), jnp.int32),
        pltpu.VMEM((2, CCH), jnp.int32),
        pltpu.VMEM((2, CCH), jnp.int32),
        pltpu.VMEM((2, CCH), jnp.int32),
        pltpu.VMEM((CCH,), jnp.float32),
        pltpu.SemaphoreType.DMA,
        pltpu.SemaphoreType.DMA,
        pltpu.SemaphoreType.DMA,
        pltpu.SemaphoreType.DMA,
    ],
)
def _pass_c(stage_hbm, srcs_hbm, dsts_hbm, rels_hbm, offs_hbm, ones_hbm,
            rows2, sb2, db2, rb2, ob2, onesb, semI0, semI1, semO0, semO1):
    wid = _worker_id()
    start = wid * PER_W
    c0 = jnp.zeros((16,), jnp.int32)
    c1 = jnp.full((16,), 1, jnp.int32)
    c2 = jnp.full((16,), 2, jnp.int32)
    c3 = jnp.full((16,), 3, jnp.int32)
    semI = (semI0, semI1)
    semO = (semO0, semO1)

    def fill_ones(i, _):
        onesb[pl.ds(i * 16, 16)] = jnp.ones((16,), jnp.float32)
        return 0

    lax.fori_loop(0, CCH // 16, fill_ones, 0)

    def issue_in(k):
        p = k % 2
        return pltpu.async_copy(
            stage_hbm.at[pl.ds(start + k * CCH, CCH)], rows2.at[p], semI[p])

    pend_in = {0: issue_in(0)}
    pend_out = [None, None]
    for k in range(NCC):
        p = k % 2
        pend_in.pop(k).wait()
        if k + 1 < NCC:
            pend_in[k + 1] = issue_in(k + 1)
        if pend_out[p] is not None:
            for cp in pend_out[p]:
                cp.wait()
        rows = rows2.at[p]
        sb, db, rb, ob = sb2.at[p], db2.at[p], rb2.at[p], ob2.at[p]

        def do_vec(i, _, rows=rows, sb=sb, db=db, rb=rb, ob=ob):
            rowi = _iota16() + i * 16
            sl = pl.ds(i * 16, 16)
            sb[sl] = plsc.load_gather(rows, [rowi, c0])
            db[sl] = plsc.load_gather(rows, [rowi, c1])
            rb[sl] = plsc.load_gather(rows, [rowi, c2])
            ob[sl] = plsc.load_gather(rows, [rowi, c3])
            return 0

        lax.fori_loop(0, CCH // 16, do_vec, 0)
        out_sl = pl.ds(start + k * CCH, CCH)
        pend_out[p] = [
            pltpu.async_copy(sb, srcs_hbm.at[out_sl], semO[p]),
            pltpu.async_copy(db, dsts_hbm.at[out_sl], semO[p]),
            pltpu.async_copy(rb, rels_hbm.at[out_sl], semO[p]),
            pltpu.async_copy(ob, offs_hbm.at[out_sl], semO[p]),
            pltpu.async_copy(onesb, ones_hbm.at[out_sl], semO[p]),
        ]
    for p in (0, 1):
        if pend_out[p] is not None:
            for cp in pend_out[p]:
                cp.wait()


def kernel(x, edges1, edges2, node2graph, num_relation1, num_relation2):
    ptable = lax.bitcast_convert_type(
        node2graph.astype(jnp.int8).reshape(PT_WORDS, 4), jnp.int32)
    src = jnp.concatenate([edges1[:, 0], edges2[:, 0]])
    dst = jnp.concatenate([edges1[:, 1], edges2[:, 1]])
    rel = jnp.concatenate(
        [edges1[:, 2], edges2[:, 2] + jnp.asarray(num_relation1, jnp.int32)])
    hist_e, hist_n, keys_all = _pass_a(src, ptable, node2graph)
    staging, num_edges = _pass_b(src, dst, rel, keys_all, hist_e, hist_n)
    src_s, dst_s, rel_s, offsets, edge_weight = _pass_c(staging)
    edge_list = jnp.stack([src_s, dst_s, rel_s], axis=1)
    return (x, edge_list, edge_weight, num_edges, offsets)


# speedup vs baseline: 1.7149x; 1.0797x over previous
import functools

import jax
import jax.numpy as jnp
from jax import lax
from jax.experimental import pallas as pl
from jax.experimental.pallas import tpu as pltpu
from jax.experimental.pallas import tpu_sc as plsc

N = 100000
E1 = 800000
E2 = 800000
E = E1 + E2
B = 64
NW = 32
PER_W = E // NW
CHUNK = 4992
NDMA = CHUNK // 128
NCH = PER_W // CHUNK
TAIL = PER_W - NCH * CHUNK
PT_WORDS = N // 4
NODE_W = 25
NODE_PER_W = N // NODE_W
CCH = 2000
NCC = PER_W // CCH

_mesh = plsc.VectorSubcoreMesh(core_axis_name="c", subcore_axis_name="s")
_params = pltpu.CompilerParams(
    needs_layout_passes=False, use_tc_tiling_on_sc=False)

_iota16 = lambda: lax.iota(jnp.int32, 16)


def _worker_id():
    return lax.axis_index("c") * 16 + lax.axis_index("s")


def _hist_update(hist, key):
    cnt, last = plsc.scan_count(key)
    h = plsc.load_gather(hist, [key])
    plsc.store_scatter(hist, [key], h + cnt, mask=last)


@functools.partial(
    pl.kernel,
    out_type=(
        jax.ShapeDtypeStruct((NW * B,), jnp.int32),
        jax.ShapeDtypeStruct((NW * B,), jnp.int32),
        jax.ShapeDtypeStruct((E,), jnp.int32),
    ),
    mesh=_mesh,
    compiler_params=_params,
    scratch_types=[
        pltpu.VMEM((PT_WORDS,), jnp.int32),
        pltpu.VMEM((2, CHUNK), jnp.int32),
        pltpu.VMEM((2, CHUNK), jnp.int32),
        pltpu.VMEM((4 * B,), jnp.int32),
        pltpu.VMEM((B,), jnp.int32),
        pltpu.VMEM((B,), jnp.int32),
        pltpu.VMEM((NODE_PER_W,), jnp.int32),
        pltpu.SemaphoreType.DMA,
        pltpu.SemaphoreType.DMA,
        pltpu.SemaphoreType.DMA,
        pltpu.SemaphoreType.DMA,
    ],
)
def _pass_a(src_hbm, ptable_hbm, n2g_hbm,
            hist_e_hbm, hist_n_hbm, keys_hbm,
            ptable, srcbuf2, keybuf2, hist4, hist_e, hist_n, nodebuf,
            semI0, semI1, semK0, semK1):
    wid = _worker_id()
    start = wid * PER_W
    semI = (semI0, semI1)
    semK = (semK0, semK1)
    pltpu.sync_copy(ptable_hbm, ptable)
    z = jnp.zeros((16,), jnp.int32)
    for j in range(4 * B // 16):
        hist4[pl.ds(j * 16, 16)] = z
    for j in range(B // 16):
        hist_n[pl.ds(j * 16, 16)] = z

    def make_do_group(sb, kb):
        def do_group(g, _):
            for q in range(4):
                i = g * 4 + q
                sl = pl.ds(i * 16, 16)
                src = sb[sl]
                word = plsc.load_gather(
                    ptable, [lax.shift_right_logical(src, 2)])
                key = lax.shift_right_logical(word, (src & 3) * 8) & 255
                kb[sl] = key
                cnt, last = plsc.scan_count(key)
                hidx = key + q * B
                h = plsc.load_gather(hist4, [hidx])
                plsc.store_scatter(hist4, [hidx], h + cnt, mask=last)
            return 0
        return do_group

    def issue_in(k):
        p = k % 2
        return pltpu.async_copy(
            src_hbm.at[pl.ds(start + k * CHUNK, CHUNK)],
            srcbuf2.at[p], semI[p])

    pend_in = {0: issue_in(0)}
    pend_key = [None, None]
    for k in range(NCH):
        p = k % 2
        pend_in.pop(k).wait()
        if k + 1 < NCH:
            pend_in[k + 1] = issue_in(k + 1)
        if pend_key[p] is not None:
            pend_key[p].wait()
        lax.fori_loop(0, CHUNK // 64,
                      make_do_group(srcbuf2.at[p], keybuf2.at[p]), 0)
        pend_key[p] = pltpu.async_copy(
            keybuf2.at[p],
            keys_hbm.at[pl.ds(wid * PER_W + k * CHUNK, CHUNK)], semK[p])
    for p in (0, 1):
        if pend_key[p] is not None:
            pend_key[p].wait()

    pltpu.sync_copy(src_hbm.at[pl.ds(start + NCH * CHUNK, TAIL)],
                    srcbuf2.at[0, pl.ds(0, TAIL)])

    def do_tail_vec(i, _):
        src = srcbuf2.at[0][pl.ds(i * 16, 16)]
        word = plsc.load_gather(ptable, [lax.shift_right_logical(src, 2)])
        key = lax.shift_right_logical(word, (src & 3) * 8) & 255
        keybuf2.at[0][pl.ds(i * 16, 16)] = key
        _hist_update(hist4, key)
        return 0

    lax.fori_loop(0, TAIL // 16, do_tail_vec, 0)
    pltpu.sync_copy(keybuf2.at[0, pl.ds(0, TAIL)],
                    keys_hbm.at[pl.ds(wid * PER_W + NCH * CHUNK, TAIL)])
    for j in range(B // 16):
        s = (hist4[pl.ds(j * 16, 16)] + hist4[pl.ds(B + j * 16, 16)]
             + hist4[pl.ds(2 * B + j * 16, 16)]
             + hist4[pl.ds(3 * B + j * 16, 16)])
        hist_e[pl.ds(j * 16, 16)] = s
    pltpu.sync_copy(hist_e, hist_e_hbm.at[pl.ds(wid * B, B)])

    @pl.when(wid < NODE_W)
    def _():
        pltpu.sync_copy(n2g_hbm.at[pl.ds(wid * NODE_PER_W, NODE_PER_W)],
                        nodebuf)

        def do_nvec(i, _):
            _hist_update(hist_n, nodebuf[pl.ds(i * 16, 16)])
            return 0

        lax.fori_loop(0, NODE_PER_W // 16, do_nvec, 0)

    pltpu.sync_copy(hist_n, hist_n_hbm.at[pl.ds(wid * B, B)])


@functools.partial(
    pl.kernel,
    out_type=(
        jax.ShapeDtypeStruct((E, 8), jnp.int32),
        jax.ShapeDtypeStruct((B,), jnp.int32),
    ),
    mesh=_mesh,
    compiler_params=_params,
    scratch_types=[
        pltpu.VMEM((CHUNK,), jnp.int32),
        pltpu.VMEM((CHUNK,), jnp.int32),
        pltpu.VMEM((CHUNK,), jnp.int32),
        pltpu.VMEM((CHUNK,), jnp.int32),
        pltpu.VMEM((2, CHUNK, 8), jnp.int32),
        pltpu.VMEM((2, NDMA, 128), jnp.int32),
        pltpu.VMEM((TAIL,), jnp.int32),
        pltpu.VMEM((NW * B,), jnp.int32),
        pltpu.VMEM((B,), jnp.int32),
        pltpu.VMEM((B,), jnp.int32),
        pltpu.VMEM((B,), jnp.int32),
        pltpu.SemaphoreType.DMA,
        pltpu.SemaphoreType.DMA,
        pltpu.SemaphoreType.DMA,
    ],
)
def _pass_b(src_hbm, dst_hbm, rel_hbm, keys_hbm,
            hist_e_hbm, hist_n_hbm,
            stage_hbm, nedges_hbm,
            srcb, dstb, relb, keybuf, stage2, pos2d2, postail,
            histbuf, base, noff, nedge, semA, semB, semIn):
    wid = _worker_id()
    start = wid * PER_W

    pltpu.sync_copy(hist_e_hbm, histbuf)
    for j in range(B // 16):
        acc = jnp.zeros((16,), jnp.int32)
        mine = jnp.zeros((16,), jnp.int32)
        for w in range(NW):
            row = histbuf[pl.ds(w * B + j * 16, 16)]
            acc = acc + row
            wv = jnp.full((16,), w, jnp.int32)
            mine = mine + jnp.where(wv < wid, row, 0)
        nedge[pl.ds(j * 16, 16)] = acc
        base[pl.ds(j * 16, 16)] = mine

    @pl.when(wid == 0)
    def _():
        pltpu.sync_copy(nedge, nedges_hbm)

    carry = jnp.zeros((), jnp.int32)
    for j in range(B // 16):
        v = nedge[pl.ds(j * 16, 16)]
        inc = plsc.cumsum(v)
        base[pl.ds(j * 16, 16)] = base[pl.ds(j * 16, 16)] + inc - v + carry
        carry = carry + jnp.sum(v)

    pltpu.sync_copy(hist_n_hbm, histbuf)
    carry = jnp.zeros((), jnp.int32)
    for j in range(B // 16):
        acc = jnp.zeros((16,), jnp.int32)
        for w in range(NODE_W):
            acc = acc + histbuf[pl.ds(w * B + j * 16, 16)]
        inc = plsc.cumsum(acc)
        noff[pl.ds(j * 16, 16)] = inc - acc + carry
        carry = carry + jnp.sum(acc)

    c0 = jnp.zeros((16,), jnp.int32)
    c1 = jnp.full((16,), 1, jnp.int32)
    c2 = jnp.full((16,), 2, jnp.int32)
    c3 = jnp.full((16,), 3, jnp.int32)

    def make_place_vec(stage):
        def place_vec(i):
            rowi = _iota16() + i * 16
            sl = pl.ds(i * 16, 16)
            key = keybuf[sl]
            cnt, last = plsc.scan_count(key)
            b = plsc.load_gather(base, [key])
            pos = b + cnt - 1
            plsc.store_scatter(base, [key], b + cnt, mask=last)
            plsc.store_scatter(stage, [rowi, c0], srcb[sl])
            plsc.store_scatter(stage, [rowi, c1], dstb[sl])
            plsc.store_scatter(stage, [rowi, c2], relb[sl])
            plsc.store_scatter(stage, [rowi, c3],
                               plsc.load_gather(noff, [key]))
            return pos
        return place_vec

    def load_cols(cstart, n, kstart, kn):
        cps = [
            pltpu.async_copy(src_hbm.at[pl.ds(cstart, n)],
                             srcb.at[pl.ds(0, n)], semIn),
            pltpu.async_copy(dst_hbm.at[pl.ds(cstart, n)],
                             dstb.at[pl.ds(0, n)], semIn),
            pltpu.async_copy(rel_hbm.at[pl.ds(cstart, n)],
                             relb.at[pl.ds(0, n)], semIn),
            pltpu.async_copy(keys_hbm.at[pl.ds(kstart, kn)],
                             keybuf.at[pl.ds(0, kn)], semIn),
        ]
        for cp in cps:
            cp.wait()

    pending = [None, None]
    for k in range(NCH):
        p = k % 2
        stage = stage2.at[p]
        sem = semA if p == 0 else semB
        if pending[p] is not None:
            for cp in pending[p]:
                cp.wait()
        load_cols(start + k * CHUNK, CHUNK,
                  wid * PER_W + k * CHUNK, CHUNK)
        place_vec = make_place_vec(stage)

        def do_vec(i, _, place_vec=place_vec, p=p):
            pos = place_vec(i)
            r = i // 8
            col = (i % 8) * 16
            plsc.store_scatter(pos2d2.at[p],
                               [jnp.full((16,), r, jnp.int32),
                                col + _iota16()], pos)
            return 0

        lax.fori_loop(0, CHUNK // 16, do_vec, 0)
        pending[p] = [pltpu.async_copy(stage.at[pl.ds(j * 128, 128)],
                                       stage_hbm.at[pos2d2.at[p, j]], sem)
                      for j in range(NDMA)]

    for p in (0, 1):
        if pending[p] is not None:
            for cp in pending[p]:
                cp.wait()
    load_cols(start + NCH * CHUNK, TAIL,
              wid * PER_W + NCH * CHUNK, TAIL)
    tail_place = make_place_vec(stage2.at[0])

    def do_tail_vec(i, _):
        postail[pl.ds(i * 16, 16)] = tail_place(i)
        return 0

    lax.fori_loop(0, TAIL // 16, do_tail_vec, 0)
    pltpu.async_copy(stage2.at[0, pl.ds(0, TAIL)],
                     stage_hbm.at[postail], semA).wait()


@functools.partial(
    pl.kernel,
    out_type=(
        jax.ShapeDtypeStruct((E,), jnp.int32),
        jax.ShapeDtypeStruct((E,), jnp.int32),
        jax.ShapeDtypeStruct((E,), jnp.int32),
        jax.ShapeDtypeStruct((E,), jnp.int32),
        jax.ShapeDtypeStruct((E,), jnp.float32),
    ),
    mesh=_mesh,
    compiler_params=_params,
    scratch_types=[
        pltpu.VMEM((2, CCH, 8), jnp.int32),
        pltpu.VMEM((2, CCH), jnp.int32),
        pltpu.VMEM((2, CCH), jnp.int32),
        pltpu.VMEM((2, CCH), jnp.int32),
        pltpu.VMEM((2, CCH), jnp.int32),
        pltpu.VMEM((CCH,), jnp.float32),
        pltpu.SemaphoreType.DMA,
        pltpu.SemaphoreType.DMA,
        pltpu.SemaphoreType.DMA,
        pltpu.SemaphoreType.DMA,
    ],
)
def _pass_c(stage_hbm, srcs_hbm, dsts_hbm, rels_hbm, offs_hbm, ones_hbm,
            rows2, sb2, db2, rb2, ob2, onesb, semI0, semI1, semO0, semO1):
    wid = _worker_id()
    start = wid * PER_W
    c0 = jnp.zeros((16,), jnp.int32)
    c1 = jnp.full((16,), 1, jnp.int32)
    c2 = jnp.full((16,), 2, jnp.int32)
    c3 = jnp.full((16,), 3, jnp.int32)
    semI = (semI0, semI1)
    semO = (semO0, semO1)

    def fill_ones(i, _):
        onesb[pl.ds(i * 16, 16)] = jnp.ones((16,), jnp.float32)
        return 0

    lax.fori_loop(0, CCH // 16, fill_ones, 0)

    def issue_in(k):
        p = k % 2
        return pltpu.async_copy(
            stage_hbm.at[pl.ds(start + k * CCH, CCH)], rows2.at[p], semI[p])

    pend_in = {0: issue_in(0)}
    pend_out = [None, None]
    for k in range(NCC):
        p = k % 2
        pend_in.pop(k).wait()
        if k + 1 < NCC:
            pend_in[k + 1] = issue_in(k + 1)
        if pend_out[p] is not None:
            for cp in pend_out[p]:
                cp.wait()
        rows = rows2.at[p]
        sb, db, rb, ob = sb2.at[p], db2.at[p], rb2.at[p], ob2.at[p]

        def do_vec(i, _, rows=rows, sb=sb, db=db, rb=rb, ob=ob):
            rowi = _iota16() + i * 16
            sl = pl.ds(i * 16, 16)
            sb[sl] = plsc.load_gather(rows, [rowi, c0])
            db[sl] = plsc.load_gather(rows, [rowi, c1])
            rb[sl] = plsc.load_gather(rows, [rowi, c2])
            ob[sl] = plsc.load_gather(rows, [rowi, c3])
            return 0

        lax.fori_loop(0, CCH // 16, do_vec, 0)
        out_sl = pl.ds(start + k * CCH, CCH)
        pend_out[p] = [
            pltpu.async_copy(sb, srcs_hbm.at[out_sl], semO[p]),
            pltpu.async_copy(db, dsts_hbm.at[out_sl], semO[p]),
            pltpu.async_copy(rb, rels_hbm.at[out_sl], semO[p]),
            pltpu.async_copy(ob, offs_hbm.at[out_sl], semO[p]),
            pltpu.async_copy(onesb, ones_hbm.at[out_sl], semO[p]),
        ]
    for p in (0, 1):
        if pend_out[p] is not None:
            for cp in pend_out[p]:
                cp.wait()


def kernel(x, edges1, edges2, node2graph, num_relation1, num_relation2):
    ptable = lax.bitcast_convert_type(
        node2graph.astype(jnp.int8).reshape(PT_WORDS, 4), jnp.int32)
    src = jnp.concatenate([edges1[:, 0], edges2[:, 0]])
    dst = jnp.concatenate([edges1[:, 1], edges2[:, 1]])
    rel = jnp.concatenate(
        [edges1[:, 2], edges2[:, 2] + jnp.asarray(num_relation1, jnp.int32)])
    hist_e, hist_n, keys_all = _pass_a(src, ptable, node2graph)
    staging, num_edges = _pass_b(src, dst, rel, keys_all, hist_e, hist_n)
    src_s, dst_s, rel_s, offsets, edge_weight = _pass_c(staging)
    edge_list = jnp.stack([src_s, dst_s, rel_s], axis=1)
    return (x, edge_list, edge_weight, num_edges, offsets)
